# baseline (speedup 1.0000x reference)
import dataclasses
import functools

import jax
import jax.numpy as jnp
import numpy as np
from jax import lax
from jax.experimental import pallas as pl
from jax.experimental.pallas import tpu as pltpu
from jax.experimental.pallas import tpu_sc as plsc

N = 10000
E = 320000
K = 5000
NT = 32
RANGE = 320
NPAD = NT * RANGE
CAP = 12800
ECH = 8000
GCH = 128
NEG = -3.0e38
LS = RANGE + 16

_mesh = plsc.VectorSubcoreMesh(core_axis_name="c", subcore_axis_name="s")

_sc_params = pltpu.CompilerParams()
if "needs_layout_passes" in pltpu.CompilerParams.__dataclass_fields__:
    _sc_params = dataclasses.replace(_sc_params, needs_layout_passes=False)


def _popc(m):
    return lax.reduce_max(plsc.all_reduce_population_count(m), axes=(0,))


def _sc1_body(src_hbm, dst_hbm, h_hbm, xq_hbm, csrc_hbm, cdl_hbm, cnt_hbm,
              ebs, ebd, csrc, cdl, lflag, acc, stage, stage2, cntv,
              sem, sem2):
    wid = lax.axis_index("c") * 16 + lax.axis_index("s")
    lo = wid * RANGE
    rlen = jnp.minimum(RANGE, N - lo)

    @pl.loop(0, RANGE, step=16)
    def _(i):
        lflag[pl.ds(i, 16)] = jnp.zeros((16,), jnp.int32)

    @pl.loop(0, RANGE + 1, step=1)
    def _(r):
        @pl.loop(0, 128, step=16)
        def _(c):
            acc[r, pl.ds(c, 16)] = jnp.full((16,), NEG, jnp.float32)

    ones = jnp.ones((16,), jnp.int32)

    def chunk(ci, pos):
        pltpu.sync_copy(src_hbm.at[pl.ds(ci * ECH, ECH)], ebs)
        pltpu.sync_copy(dst_hbm.at[pl.ds(ci * ECH, ECH)], ebd)

        def vec(j, pos):
            d = ebd[pl.ds(j * 16, 16)]
            s = ebs[pl.ds(j * 16, 16)]
            dl = d - lo
            m = (dl >= 0) & (dl < rlen) & (pos < CAP - 16)
            mloop = m & (s == d)
            plsc.store_scatter(lflag, [jnp.where(mloop, dl, 0)], ones,
                               mask=mloop)
            plsc.store_compressed(csrc.at[pl.ds(pos, 16)], s, mask=m)
            plsc.store_compressed(cdl.at[pl.ds(pos, 16)], dl, mask=m)
            return pos + _popc(m)

        return lax.fori_loop(0, ECH // 16, vec, pos)

    pos = lax.fori_loop(0, E // ECH, chunk, 0)

    def app(j, pos):
        dl = lax.iota(jnp.int32, 16) + j * 16
        flg = lflag[pl.ds(j * 16, 16)]
        m = (dl < rlen) & (flg == 0)
        plsc.store_compressed(csrc.at[pl.ds(pos, 16)], dl + lo, mask=m)
        plsc.store_compressed(cdl.at[pl.ds(pos, 16)], dl, mask=m)
        return pos + _popc(m)

    pos = lax.fori_loop(0, RANGE // 16, app, pos)

    npad = (pos + (2 * GCH - 1)) & ~(2 * GCH - 1)

    def fill(j, _):
        csrc[pl.ds(pos + j * 16, 16)] = jnp.zeros((16,), jnp.int32)
        cdl[pl.ds(pos + j * 16, 16)] = jnp.full((16,), RANGE, jnp.int32)
        return 0

    lax.fori_loop(0, (npad - pos + 15) // 16, fill, 0)

    def pair(p, _):
        b0 = p * 2 * GCH
        c0 = pltpu.async_copy(h_hbm.at[csrc.at[pl.ds(b0, GCH)]], stage, sem)
        c1 = pltpu.async_copy(h_hbm.at[csrc.at[pl.ds(b0 + GCH, GCH)]],
                              stage2, sem2)
        c0.wait()

        def row0(r, _):
            dl = cdl[pl.ds(b0 + r, 16)][0]
            for c in range(8):
                a = acc[dl, pl.ds(c * 16, 16)]
                b = stage[r, pl.ds(c * 16, 16)]
                acc[dl, pl.ds(c * 16, 16)] = jnp.maximum(a, b)
            return 0

        lax.fori_loop(0, GCH, row0, 0)
        c1.wait()

        def row1(r, _):
            dl = cdl[pl.ds(b0 + GCH + r, 16)][0]
            for c in range(8):
                a = acc[dl, pl.ds(c * 16, 16)]
                b = stage2[r, pl.ds(c * 16, 16)]
                acc[dl, pl.ds(c * 16, 16)] = jnp.maximum(a, b)
            return 0

        lax.fori_loop(0, GCH, row1, 0)
        return 0

    lax.fori_loop(0, npad // (2 * GCH), pair, 0)

    pltpu.sync_copy(acc.at[pl.ds(0, RANGE)], xq_hbm.at[pl.ds(lo, RANGE)])
    pltpu.sync_copy(csrc.at[pl.ds(0, CAP)], csrc_hbm.at[wid])
    pltpu.sync_copy(cdl.at[pl.ds(0, CAP)], cdl_hbm.at[wid])
    cntv[...] = jnp.full((16,), 0, jnp.int32) + npad
    pltpu.sync_copy(cntv, cnt_hbm.at[wid])


@jax.jit
def _sc1(src, dst, h):
    f = pl.kernel(
        _sc1_body,
        out_type=[
            jax.ShapeDtypeStruct((NPAD, 128), jnp.float32),
            jax.ShapeDtypeStruct((NT, CAP), jnp.int32),
            jax.ShapeDtypeStruct((NT, CAP), jnp.int32),
            jax.ShapeDtypeStruct((NT, 16), jnp.int32),
        ],
        mesh=_mesh,
        compiler_params=_sc_params,
        scratch_types=[
            pltpu.VMEM((ECH,), jnp.int32),
            pltpu.VMEM((ECH,), jnp.int32),
            pltpu.VMEM((CAP + 16,), jnp.int32),
            pltpu.VMEM((CAP + 16,), jnp.int32),
            pltpu.VMEM((RANGE,), jnp.int32),
            pltpu.VMEM((RANGE + 1, 128), jnp.float32),
            pltpu.VMEM((GCH, 128), jnp.float32),
            pltpu.VMEM((GCH, 128), jnp.float32),
            pltpu.VMEM((16,), jnp.int32),
            pltpu.SemaphoreType.DMA,
            pltpu.SemaphoreType.DMA,
        ],
    )
    return f(src, dst, h)


def _sc2_body(csrc_hbm, cdl_hbm, cnt_hbm, h_hbm, beta_hbm, s1_hbm, mm_hbm,
              xnew_hbm, deg_hbm,
              csrc, cdl, btc, s1v, mmv, ssr, degr, ss16, acc, stage, stage2,
              wnc, cntv, sem, sem2):
    wid = lax.axis_index("c") * 16 + lax.axis_index("s")
    lo = wid * RANGE

    pltpu.sync_copy(cnt_hbm.at[wid], cntv)
    npad = cntv[...][0]

    pltpu.sync_copy(csrc_hbm.at[wid], csrc.at[pl.ds(0, CAP)])
    pltpu.sync_copy(cdl_hbm.at[wid], cdl.at[pl.ds(0, CAP)])
    pltpu.sync_copy(s1_hbm.at[pl.ds(lo, RANGE)], s1v.at[pl.ds(0, RANGE)])
    pltpu.sync_copy(mm_hbm.at[pl.ds(lo, RANGE)], mmv.at[pl.ds(0, RANGE)])
    s1v[pl.ds(RANGE, 16)] = jnp.zeros((16,), jnp.float32)
    mmv[pl.ds(RANGE, 16)] = jnp.zeros((16,), jnp.float32)

    def bchunk(g, _):
        pltpu.async_copy(beta_hbm.at[csrc.at[pl.ds(g * GCH, GCH)]],
                         btc.at[pl.ds(g * GCH, GCH)], sem).wait()
        return 0

    lax.fori_loop(0, npad // GCH, bchunk, 0)

    @pl.loop(0, 32 * LS, step=16)
    def _(i):
        ss16[pl.ds(i, 16)] = jnp.zeros((16,), jnp.float32)

    @pl.loop(0, RANGE + 1, step=1)
    def _(r):
        @pl.loop(0, 128, step=16)
        def _(c):
            acc[r, pl.ds(c, 16)] = jnp.zeros((16,), jnp.float32)

    lanes = lax.iota(jnp.int32, 16) * LS
    onesf = jnp.ones((16,), jnp.float32)

    def vecb(j, _):
        dl = cdl[pl.ds(j * 16, 16)]
        bt = btc[pl.ds(j * 16, 16)]
        a1 = plsc.load_gather(s1v, [dl])
        mm = plsc.load_gather(mmv, [dl])
        z = a1 + bt
        scr = jnp.where(z > 0, z, 0.2 * z)
        w = jnp.exp(scr - mm)
        plsc.addupdate_scatter(ss16, [dl + lanes], w)
        plsc.addupdate_scatter(ss16, [dl + lanes + 16 * LS], onesf)
        return 0

    lax.fori_loop(0, npad // 16, vecb, 0)

    @pl.loop(0, RANGE + 16, step=16)
    def _(i):
        t = jnp.zeros((16,), jnp.float32)
        u = jnp.zeros((16,), jnp.float32)
        for l in range(16):
            t = t + ss16[pl.ds(l * LS + i, 16)]
            u = u + ss16[pl.ds((16 + l) * LS + i, 16)]
        ssr[pl.ds(i, 16)] = t
        degr[pl.ds(i, 16)] = u

    def pair(p, _):
        b0 = p * 2 * GCH
        c0 = pltpu.async_copy(h_hbm.at[csrc.at[pl.ds(b0, GCH)]], stage, sem)
        c1 = pltpu.async_copy(h_hbm.at[csrc.at[pl.ds(b0 + GCH, GCH)]],
                              stage2, sem2)

        @pl.loop(0, 2 * GCH, step=16)
        def _(j):
            dl = cdl[pl.ds(b0 + j, 16)]
            bt = btc[pl.ds(b0 + j, 16)]
            a1 = plsc.load_gather(s1v, [dl])
            mm = plsc.load_gather(mmv, [dl])
            z = a1 + bt
            scr = jnp.where(z > 0, z, 0.2 * z)
            w = jnp.exp(scr - mm)
            ss = plsc.load_gather(ssr, [dl])
            wnc[pl.ds(j, 16)] = w / (ss + 1e-16)

        c0.wait()

        def row0(r, _):
            dl = cdl[pl.ds(b0 + r, 16)][0]
            wn = wnc[pl.ds(r, 16)][0]
            for c in range(8):
                a = acc[dl, pl.ds(c * 16, 16)]
                b = stage[r, pl.ds(c * 16, 16)]
                acc[dl, pl.ds(c * 16, 16)] = a + wn * b
            return 0

        lax.fori_loop(0, GCH, row0, 0)
        c1.wait()

        def row1(r, _):
            dl = cdl[pl.ds(b0 + GCH + r, 16)][0]
            wn = wnc[pl.ds(GCH + r, 16)][0]
            for c in range(8):
                a = acc[dl, pl.ds(c * 16, 16)]
                b = stage2[r, pl.ds(c * 16, 16)]
                acc[dl, pl.ds(c * 16, 16)] = a + wn * b
            return 0

        lax.fori_loop(0, GCH, row1, 0)
        return 0

    lax.fori_loop(0, npad // (2 * GCH), pair, 0)

    pltpu.sync_copy(acc.at[pl.ds(0, RANGE)], xnew_hbm.at[pl.ds(lo, RANGE)])
    pltpu.sync_copy(degr.at[pl.ds(0, RANGE)], deg_hbm.at[pl.ds(lo, RANGE)])


@jax.jit
def _sc2(csrc, cdl, cnt, h, beta, s1, mm):
    f = pl.kernel(
        _sc2_body,
        out_type=[
            jax.ShapeDtypeStruct((NPAD, 128), jnp.float32),
            jax.ShapeDtypeStruct((NPAD,), jnp.float32),
        ],
        mesh=_mesh,
        compiler_params=_sc_params,
        scratch_types=[
            pltpu.VMEM((CAP + 16,), jnp.int32),
            pltpu.VMEM((CAP + 16,), jnp.int32),
            pltpu.VMEM((CAP + 16,), jnp.float32),
            pltpu.VMEM((RANGE + 16,), jnp.float32),
            pltpu.VMEM((RANGE + 16,), jnp.float32),
            pltpu.VMEM((RANGE + 16,), jnp.float32),
            pltpu.VMEM((RANGE + 16,), jnp.float32),
            pltpu.VMEM((32 * LS,), jnp.float32),
            pltpu.VMEM((RANGE + 1, 128), jnp.float32),
            pltpu.VMEM((GCH, 128), jnp.float32),
            pltpu.VMEM((GCH, 128), jnp.float32),
            pltpu.VMEM((2 * GCH + 16,), jnp.float32),
            pltpu.VMEM((16,), jnp.int32),
            pltpu.SemaphoreType.DMA,
            pltpu.SemaphoreType.DMA,
        ],
    )
    return f(csrc, cdl, cnt, h, beta, s1, mm)


def _sc3_body(csrc_hbm, cdl_hbm, cnt_hbm, g1a_hbm, asum_hbm,
              csrc, cdl, gac, as16, red, cntv, sem):
    wid = lax.axis_index("c") * 16 + lax.axis_index("s")
    lo = wid * RANGE

    pltpu.sync_copy(cnt_hbm.at[wid], cntv)
    npad = cntv[...][0]
    pltpu.sync_copy(csrc_hbm.at[wid], csrc.at[pl.ds(0, CAP)])
    pltpu.sync_copy(cdl_hbm.at[wid], cdl.at[pl.ds(0, CAP)])

    def bchunk(g, _):
        pltpu.async_copy(g1a_hbm.at[csrc.at[pl.ds(g * GCH, GCH)]],
                         gac.at[pl.ds(g * GCH, GCH)], sem).wait()
        return 0

    lax.fori_loop(0, npad // GCH, bchunk, 0)

    @pl.loop(0, 16 * LS, step=16)
    def _(i):
        as16[pl.ds(i, 16)] = jnp.zeros((16,), jnp.float32)

    lanes = lax.iota(jnp.int32, 16) * LS

    def veca(j, _):
        dl = cdl[pl.ds(j * 16, 16)]
        ga = gac[pl.ds(j * 16, 16)]
        plsc.addupdate_scatter(as16, [dl + lanes], ga)
        return 0

    lax.fori_loop(0, npad // 16, veca, 0)

    @pl.loop(0, RANGE + 16, step=16)
    def _(i):
        t = jnp.zeros((16,), jnp.float32)
        for l in range(16):
            t = t + as16[pl.ds(l * LS + i, 16)]
        red[pl.ds(i, 16)] = t

    pltpu.sync_copy(red.at[pl.ds(0, RANGE)], asum_hbm.at[pl.ds(lo, RANGE)])


@jax.jit
def _sc3(csrc, cdl, cnt, g1a):
    f = pl.kernel(
        _sc3_body,
        out_type=jax.ShapeDtypeStruct((NPAD,), jnp.float32),
        mesh=_mesh,
        compiler_params=_sc_params,
        scratch_types=[
            pltpu.VMEM((CAP + 16,), jnp.int32),
            pltpu.VMEM((CAP + 16,), jnp.int32),
            pltpu.VMEM((CAP + 16,), jnp.float32),
            pltpu.VMEM((16 * LS,), jnp.float32),
            pltpu.VMEM((RANGE + 16,), jnp.float32),
            pltpu.VMEM((16,), jnp.int32),
            pltpu.SemaphoreType.DMA,
        ],
    )
    return f(csrc, cdl, cnt, g1a)


def _hb_body(x_ref, w_ref, b_ref, wa2_ref, h_ref, beta_ref, bmax_ref):
    i = pl.program_id(0)
    h = jax.nn.relu(
        lax.dot_general(x_ref[...], w_ref[...], (((1,), (0,)), ((), ())),
                        preferred_element_type=jnp.float32) + b_ref[...])
    h_ref[...] = h
    beta = jnp.sum(h * wa2_ref[...], axis=1)
    beta_ref[...] = beta[:, None]
    bm = jnp.max(beta)
    prev = jnp.where(i == 0, jnp.float32(NEG), bmax_ref[...][0, 0])
    bmax_ref[...] = jnp.reshape(jnp.maximum(prev, bm), (1, 1))


@jax.jit
def _k_h(x, W1, b1, wa2):
    blk = 400
    return pl.pallas_call(
        _hb_body,
        grid=(N // blk,),
        in_specs=[
            pl.BlockSpec((blk, 128), lambda i: (i, 0)),
            pl.BlockSpec((128, 128), lambda i: (0, 0)),
            pl.BlockSpec((1, 128), lambda i: (0, 0)),
            pl.BlockSpec((1, 128), lambda i: (0, 0)),
        ],
        out_specs=[
            pl.BlockSpec((blk, 128), lambda i: (i, 0)),
            pl.BlockSpec((blk, 1), lambda i: (i, 0)),
            pl.BlockSpec((1, 1), lambda i: (0, 0)),
        ],
        out_shape=[
            jax.ShapeDtypeStruct((N, 128), jnp.float32),
            jax.ShapeDtypeStruct((N, 1), jnp.float32),
            jax.ShapeDtypeStruct((1, 1), jnp.float32),
        ],
    )(x, W1, b1[None, :], wa2[None, :])


def _alpha_body(xq_ref, u_ref, c0_ref, bmax_ref, s1_ref, mm_ref):
    s1 = jnp.sum(xq_ref[...] * u_ref[...], axis=1) + c0_ref[0, 0]
    s1_ref[...] = s1[:, None]
    z = s1 + bmax_ref[0, 0]
    mm_ref[...] = jnp.where(z > 0, z, 0.2 * z)[:, None]


@jax.jit
def _k_alpha(xq, u, c0, bmax):
    blk = 512
    return pl.pallas_call(
        _alpha_body,
        grid=(NPAD // blk,),
        in_specs=[
            pl.BlockSpec((blk, 128), lambda i: (i, 0)),
            pl.BlockSpec((1, 128), lambda i: (0, 0)),
            pl.BlockSpec((1, 1), lambda i: (0, 0)),
            pl.BlockSpec((1, 1), lambda i: (0, 0)),
        ],
        out_specs=[
            pl.BlockSpec((blk, 1), lambda i: (i, 0)),
            pl.BlockSpec((blk, 1), lambda i: (i, 0)),
        ],
        out_shape=[
            jax.ShapeDtypeStruct((NPAD, 1), jnp.float32),
            jax.ShapeDtypeStruct((NPAD, 1), jnp.float32),
        ],
    )(xq, u[None, :], c0, bmax)


def _g_body(x_ref, wg_ref, bg_ref, g1a_ref, gb_ref, g3_ref):
    x = x_ref[...]
    g1a_ref[...] = (jnp.sum(x * wg_ref[0:1, :], axis=1) + bg_ref[0, 0])[:, None]
    gb_ref[...] = jnp.sum(x * wg_ref[1:2, :], axis=1)[:, None]
    g3_ref[...] = (jnp.sum(x * wg_ref[2:3, :], axis=1) + bg_ref[0, 1])[:, None]


@jax.jit
def _k_g(xnew, wg3x, bgv):
    blk = 512
    return pl.pallas_call(
        _g_body,
        grid=(NPAD // blk,),
        in_specs=[
            pl.BlockSpec((blk, 128), lambda i: (i, 0)),
            pl.BlockSpec((3, 128), lambda i: (0, 0)),
            pl.BlockSpec((1, 2), lambda i: (0, 0)),
        ],
        out_specs=[
            pl.BlockSpec((blk, 1), lambda i: (i, 0)),
            pl.BlockSpec((blk, 1), lambda i: (i, 0)),
            pl.BlockSpec((blk, 1), lambda i: (i, 0)),
        ],
        out_shape=[
            jax.ShapeDtypeStruct((NPAD, 1), jnp.float32),
            jax.ShapeDtypeStruct((NPAD, 1), jnp.float32),
            jax.ShapeDtypeStruct((NPAD, 1), jnp.float32),
        ],
    )(xnew, wg3x, bgv)


def _topk_body(asum_ref, deg_ref, gb_ref, g3_ref, xnew_ref, w2_ref, b2_ref,
               o_ref):
    fit = jax.nn.sigmoid(asum_ref[...] - deg_ref[...] * gb_ref[...]
                         + g3_ref[...])
    idx = lax.broadcasted_iota(jnp.int32, (NPAD,), 0)
    fit = jnp.where(idx < N, fit, -1.0)
    bits = lax.bitcast_convert_type(fit, jnp.int32)

    def sbit(b, thr):
        cand = thr | (1 << b)
        cnt = jnp.sum(jnp.where(bits >= cand, 1, 0))
        return jnp.where(cnt >= K, cand, thr)

    thr = lax.fori_loop(0, 31, lambda i, t: sbit(30 - i, t), 0)

    c_gt = jnp.sum(jnp.where(bits > thr, 1, 0))
    t = K - c_gt
    tie = bits == thr

    def mbit(b, m):
        cand = m | (1 << b)
        g = jnp.sum(jnp.where(tie & (idx < cand), 1, 0))
        return jnp.where(g <= t, cand, m)

    m = lax.fori_loop(0, 14, lambda i, mm: mbit(13 - i, mm), 0)

    sel = (bits > thr) | (tie & (idx < m))
    w = jnp.where(sel, fit, 0.0)
    s = jnp.sum(xnew_ref[...] * w[:, None], axis=0) * (1.0 / K)
    o_ref[...] = (lax.dot_general(s[None, :], w2_ref[...],
                                  (((1,), (0,)), ((), ())),
                                  preferred_element_type=jnp.float32)
                  + b2_ref[...])


@jax.jit
def _k_topk(asum, deg, gb, g3, xnew, W2, b2):
    return pl.pallas_call(
        _topk_body,
        out_shape=jax.ShapeDtypeStruct((1, 64), jnp.float32),
    )(asum, deg, gb, g3, xnew, W2, b2[None, :])


def kernel(x, edge_index, batch, W1, b1, Wp, bp, Wa, ba, Wg1, bg1, Wg2, Wg3, bg3, W2, b2):
    src, dst = edge_index[0], edge_index[1]
    wa1 = Wa[:128, 0]
    wa2 = Wa[128:, 0]
    u = Wp @ wa1
    c0 = jnp.reshape(jnp.dot(bp, wa1) + ba[0], (1, 1))
    wg3x = jnp.stack([Wg1[:, 0], Wg2[:, 0], Wg3[:, 0]], axis=0)
    bgv = jnp.stack([bg1[0], bg3[0]]).reshape(1, 2)

    h, beta, bmax = _k_h(x, W1, b1, wa2)
    xq_pad, csrc, cdl, cnt = _sc1(src, dst, h)
    s1, mm = _k_alpha(xq_pad, u, c0, bmax)
    beta_pad = jnp.pad(beta.reshape(-1), (0, NPAD - N))
    xnew, deg = _sc2(csrc, cdl, cnt, h, beta_pad, s1.reshape(-1),
                     mm.reshape(-1))
    g1a, gb, g3 = _k_g(xnew, wg3x, bgv)
    asum = _sc3(csrc, cdl, cnt, g1a.reshape(-1))
    out = _k_topk(asum, deg, gb.reshape(-1), g3.reshape(-1), xnew, W2, b2)
    return out


# speedup vs baseline: 1.3826x; 1.3826x over previous
import dataclasses
import functools

import jax
import jax.numpy as jnp
import numpy as np
from jax import lax
from jax.experimental import pallas as pl
from jax.experimental.pallas import tpu as pltpu
from jax.experimental.pallas import tpu_sc as plsc

N = 10000
E = 320000
K = 5000
NT = 32
RANGE = 320
NPAD = NT * RANGE
CAP = 12800
ECH = 8000
GCH = 128
NEG = -3.0e38
LS = 384

_mesh = plsc.VectorSubcoreMesh(core_axis_name="c", subcore_axis_name="s")

_sc_params = pltpu.CompilerParams()
if "needs_layout_passes" in pltpu.CompilerParams.__dataclass_fields__:
    _sc_params = dataclasses.replace(_sc_params, needs_layout_passes=False)


def _popc(m):
    return lax.reduce_max(plsc.all_reduce_population_count(m), axes=(0,))


def _sc0_body(src_hbm, dst_hbm, csrc_hbm, cdl_hbm, off_hbm, cnt_hbm,
              ebs, ebd, csrc, cdl, tmps, tmpd, lflag, cnt16, cur16, offv,
              cntv):
    wid = lax.axis_index("c") * 16 + lax.axis_index("s")
    lo = wid * RANGE
    rlen = jnp.minimum(RANGE, N - lo)

    @pl.loop(0, RANGE, step=16)
    def _(i):
        lflag[pl.ds(i, 16)] = jnp.zeros((16,), jnp.int32)

    ones = jnp.ones((16,), jnp.int32)

    def chunk(ci, pos):
        pltpu.sync_copy(src_hbm.at[pl.ds(ci * ECH, ECH)], ebs)
        pltpu.sync_copy(dst_hbm.at[pl.ds(ci * ECH, ECH)], ebd)

        def vec(j, pos):
            d = ebd[pl.ds(j * 16, 16)]
            s = ebs[pl.ds(j * 16, 16)]
            dl = d - lo
            m = (dl >= 0) & (dl < rlen) & (pos < CAP - 16)
            mloop = m & (s == d)
            plsc.store_scatter(lflag, [jnp.where(mloop, dl, 0)], ones,
                               mask=mloop)
            plsc.store_compressed(csrc.at[pl.ds(pos, 16)], s, mask=m)
            plsc.store_compressed(cdl.at[pl.ds(pos, 16)], dl, mask=m)
            return pos + _popc(m)

        return lax.fori_loop(0, ECH // 16, vec, pos)

    pos = lax.fori_loop(0, E // ECH, chunk, 0)

    def app(j, pos):
        dl = lax.iota(jnp.int32, 16) + j * 16
        flg = lflag[pl.ds(j * 16, 16)]
        m = (dl < rlen) & (flg == 0) & (pos < CAP - 16)
        plsc.store_compressed(csrc.at[pl.ds(pos, 16)], dl + lo, mask=m)
        plsc.store_compressed(cdl.at[pl.ds(pos, 16)], dl, mask=m)
        return pos + _popc(m)

    pos = lax.fori_loop(0, RANGE // 16, app, pos)

    npad = (pos + (2 * GCH - 1)) & ~(2 * GCH - 1)

    def fill(j, _):
        csrc[pl.ds(pos + j * 16, 16)] = jnp.zeros((16,), jnp.int32)
        cdl[pl.ds(pos + j * 16, 16)] = jnp.full((16,), RANGE, jnp.int32)
        return 0

    lax.fori_loop(0, (npad - pos + 15) // 16, fill, 0)

    lanes = lax.iota(jnp.int32, 16) * LS

    @pl.loop(0, 16 * LS, step=16)
    def _(i):
        cnt16[pl.ds(i, 16)] = jnp.zeros((16,), jnp.int32)

    def vcount(j, _):
        dl = cdl[pl.ds(j * 16, 16)]
        plsc.addupdate_scatter(cnt16, [dl + lanes], ones)
        return 0

    lax.fori_loop(0, npad // 16, vcount, 0)

    def oblk(i, base):
        tot = jnp.zeros((16,), jnp.int32)
        for l in range(16):
            tot = tot + cnt16[pl.ds(l * LS + i * 16, 16)]
        ps = plsc.cumsum(tot)
        offv[pl.ds(i * 16, 16)] = base + (ps - tot)
        return base + ps[15]

    lax.fori_loop(0, LS // 16, oblk, 0)

    @pl.loop(0, LS, step=16)
    def _(i):
        run = offv[pl.ds(i, 16)]
        for l in range(16):
            cur16[pl.ds(l * LS + i, 16)] = run
            run = run + cnt16[pl.ds(l * LS + i, 16)]

    def vscat(j, _):
        dl = cdl[pl.ds(j * 16, 16)]
        s = csrc[pl.ds(j * 16, 16)]
        posv = plsc.load_gather(cur16, [dl + lanes])
        plsc.store_scatter(tmps, [posv], s)
        plsc.store_scatter(tmpd, [posv], dl)
        plsc.store_scatter(cur16, [dl + lanes], posv + 1)
        return 0

    lax.fori_loop(0, npad // 16, vscat, 0)

    pltpu.sync_copy(tmps.at[pl.ds(0, CAP)], csrc_hbm.at[wid])
    pltpu.sync_copy(tmpd.at[pl.ds(0, CAP)], cdl_hbm.at[wid])
    pltpu.sync_copy(offv, off_hbm.at[wid])
    cntv[...] = jnp.full((16,), 0, jnp.int32) + npad
    pltpu.sync_copy(cntv, cnt_hbm.at[wid])


@jax.jit
def _sc0(src, dst):
    f = pl.kernel(
        _sc0_body,
        out_type=[
            jax.ShapeDtypeStruct((NT, CAP), jnp.int32),
            jax.ShapeDtypeStruct((NT, CAP), jnp.int32),
            jax.ShapeDtypeStruct((NT, LS), jnp.int32),
            jax.ShapeDtypeStruct((NT, 16), jnp.int32),
        ],
        mesh=_mesh,
        compiler_params=_sc_params,
        scratch_types=[
            pltpu.VMEM((ECH,), jnp.int32),
            pltpu.VMEM((ECH,), jnp.int32),
            pltpu.VMEM((CAP + 16,), jnp.int32),
            pltpu.VMEM((CAP + 16,), jnp.int32),
            pltpu.VMEM((CAP + 16,), jnp.int32),
            pltpu.VMEM((CAP + 16,), jnp.int32),
            pltpu.VMEM((RANGE,), jnp.int32),
            pltpu.VMEM((16 * LS,), jnp.int32),
            pltpu.VMEM((16 * LS,), jnp.int32),
            pltpu.VMEM((LS,), jnp.int32),
            pltpu.VMEM((16,), jnp.int32),
        ],
    )
    return f(src, dst)


def _sc1_body(csrc_hbm, off_hbm, cnt_hbm, h_hbm, xq_hbm,
              csrc, offv, acc, stage, stage2, cntv, sem, sem2):
    wid = lax.axis_index("c") * 16 + lax.axis_index("s")
    lo = wid * RANGE
    rlen = jnp.minimum(RANGE, N - lo)

    pltpu.sync_copy(cnt_hbm.at[wid], cntv)
    npad = cntv[...][0]
    pltpu.sync_copy(csrc_hbm.at[wid], csrc.at[pl.ds(0, CAP)])
    pltpu.sync_copy(off_hbm.at[wid], offv.at[pl.ds(0, LS)])

    @pl.loop(0, 128, step=16)
    def _(c):
        z = jnp.zeros((16,), jnp.float32)

        def zr(r, _):
            acc[r, pl.ds(c, 16)] = z
            return 0

        lax.fori_loop(rlen, RANGE, zr, 0)

    negs = jnp.full((16,), NEG, jnp.float32)

    def make_edge(stg, base):
        def edge(r, carry):
            dcur, nb, r0, r1, r2, r3, r4, r5, r6, r7 = carry
            e = base + r
            flush = (e == nb) & (dcur < rlen)
            regs = [r0, r1, r2, r3, r4, r5, r6, r7]

            @pl.when(flush)
            def _():
                for c in range(8):
                    acc[dcur, pl.ds(c * 16, 16)] = regs[c]

            dcur = jnp.where(flush, dcur + 1, dcur)
            nb = jnp.where(flush, offv[pl.ds(dcur + 1, 16)][0], nb)
            out = []
            for c in range(8):
                b = stg[r, pl.ds(c * 16, 16)]
                out.append(jnp.maximum(jnp.where(flush, negs, regs[c]), b))
            return (dcur, nb, *out)

        return edge

    nb0 = offv[pl.ds(1, 16)][0]
    carry = (jnp.int32(0), nb0, *([negs] * 8))

    def pair(p, carry):
        b0 = p * 2 * GCH
        c0 = pltpu.async_copy(h_hbm.at[csrc.at[pl.ds(b0, GCH)]], stage, sem)
        c1 = pltpu.async_copy(h_hbm.at[csrc.at[pl.ds(b0 + GCH, GCH)]],
                              stage2, sem2)
        c0.wait()
        carry = lax.fori_loop(0, GCH, make_edge(stage, b0), carry)
        c1.wait()
        carry = lax.fori_loop(0, GCH, make_edge(stage2, b0 + GCH), carry)
        return carry

    carry = lax.fori_loop(0, npad // (2 * GCH), pair, carry)

    dcur = carry[0]
    regs = carry[2:]

    @pl.when(dcur < rlen)
    def _():
        for c in range(8):
            acc[dcur, pl.ds(c * 16, 16)] = regs[c]

    pltpu.sync_copy(acc.at[pl.ds(0, RANGE)], xq_hbm.at[pl.ds(lo, RANGE)])


@jax.jit
def _sc1(csrc, off, cnt, h):
    f = pl.kernel(
        _sc1_body,
        out_type=jax.ShapeDtypeStruct((NPAD, 128), jnp.float32),
        mesh=_mesh,
        compiler_params=_sc_params,
        scratch_types=[
            pltpu.VMEM((CAP + 16,), jnp.int32),
            pltpu.VMEM((LS + 16,), jnp.int32),
            pltpu.VMEM((RANGE, 128), jnp.float32),
            pltpu.VMEM((GCH, 128), jnp.float32),
            pltpu.VMEM((GCH, 128), jnp.float32),
            pltpu.VMEM((16,), jnp.int32),
            pltpu.SemaphoreType.DMA,
            pltpu.SemaphoreType.DMA,
        ],
    )
    return f(csrc, off, cnt, h)


def _sc2_body(csrc_hbm, cdl_hbm, off_hbm, cnt_hbm, h_hbm, beta_hbm, s1_hbm,
              mm_hbm, xnew_hbm, deg_hbm,
              csrc, cdl, wv, btc, s1v, mmv, ssr, degr, ss16, offv, acc,
              stage, stage2, wnc, cntv, sem, sem2):
    wid = lax.axis_index("c") * 16 + lax.axis_index("s")
    lo = wid * RANGE
    rlen = jnp.minimum(RANGE, N - lo)

    pltpu.sync_copy(cnt_hbm.at[wid], cntv)
    npad = cntv[...][0]

    pltpu.sync_copy(csrc_hbm.at[wid], csrc.at[pl.ds(0, CAP)])
    pltpu.sync_copy(cdl_hbm.at[wid], cdl.at[pl.ds(0, CAP)])
    pltpu.sync_copy(off_hbm.at[wid], offv.at[pl.ds(0, LS)])
    pltpu.sync_copy(s1_hbm.at[pl.ds(lo, RANGE)], s1v.at[pl.ds(0, RANGE)])
    pltpu.sync_copy(mm_hbm.at[pl.ds(lo, RANGE)], mmv.at[pl.ds(0, RANGE)])
    s1v[pl.ds(RANGE, 16)] = jnp.zeros((16,), jnp.float32)
    mmv[pl.ds(RANGE, 16)] = jnp.zeros((16,), jnp.float32)

    @pl.loop(0, 16 * LS, step=16)
    def _(i):
        ss16[pl.ds(i, 16)] = jnp.zeros((16,), jnp.float32)

    lanes = lax.iota(jnp.int32, 16) * LS

    def bchunk(g, _):
        pltpu.async_copy(beta_hbm.at[csrc.at[pl.ds(g * GCH, GCH)]],
                         btc, sem).wait()

        @pl.loop(0, GCH, step=16)
        def _(j):
            dl = cdl[pl.ds(g * GCH + j, 16)]
            bt = btc[pl.ds(j, 16)]
            a1 = plsc.load_gather(s1v, [dl])
            mm = plsc.load_gather(mmv, [dl])
            z = a1 + bt
            scr = jnp.where(z > 0, z, 0.2 * z)
            w = jnp.exp(scr - mm)
            wv[pl.ds(g * GCH + j, 16)] = w
            plsc.addupdate_scatter(ss16, [dl + lanes], w)

        return 0

    lax.fori_loop(0, npad // GCH, bchunk, 0)

    @pl.loop(0, RANGE + 16, step=16)
    def _(i):
        t = jnp.zeros((16,), jnp.float32)
        for l in range(16):
            t = t + ss16[pl.ds(l * LS + i, 16)]
        ssr[pl.ds(i, 16)] = t

    @pl.loop(0, RANGE, step=16)
    def _(i):
        d0 = offv[pl.ds(i, 16)]
        d1 = offv[pl.ds(i + 1, 16)]
        degr[pl.ds(i, 16)] = (d1 - d0).astype(jnp.float32)

    @pl.loop(0, 128, step=16)
    def _(c):
        z = jnp.zeros((16,), jnp.float32)

        def zr(r, _):
            acc[r, pl.ds(c, 16)] = z
            return 0

        lax.fori_loop(rlen, RANGE, zr, 0)

    zeros = jnp.zeros((16,), jnp.float32)

    def make_edge(stg, base, wbase):
        def edge(r, carry):
            dcur, nb, r0, r1, r2, r3, r4, r5, r6, r7 = carry
            e = base + r
            flush = (e == nb) & (dcur < rlen)
            regs = [r0, r1, r2, r3, r4, r5, r6, r7]

            @pl.when(flush)
            def _():
                for c in range(8):
                    acc[dcur, pl.ds(c * 16, 16)] = regs[c]

            dcur = jnp.where(flush, dcur + 1, dcur)
            nb = jnp.where(flush, offv[pl.ds(dcur + 1, 16)][0], nb)
            wn = wnc[pl.ds(wbase + r, 16)][0]
            out = []
            for c in range(8):
                b = stg[r, pl.ds(c * 16, 16)]
                out.append(jnp.where(flush, zeros, regs[c]) + wn * b)
            return (dcur, nb, *out)

        return edge

    nb0 = offv[pl.ds(1, 16)][0]
    carry = (jnp.int32(0), nb0, *([zeros] * 8))

    def pair(p, carry):
        b0 = p * 2 * GCH
        c0 = pltpu.async_copy(h_hbm.at[csrc.at[pl.ds(b0, GCH)]], stage, sem)
        c1 = pltpu.async_copy(h_hbm.at[csrc.at[pl.ds(b0 + GCH, GCH)]],
                              stage2, sem2)

        @pl.loop(0, 2 * GCH, step=16)
        def _(j):
            dl = cdl[pl.ds(b0 + j, 16)]
            w = wv[pl.ds(b0 + j, 16)]
            ss = plsc.load_gather(ssr, [dl])
            wnc[pl.ds(j, 16)] = w / (ss + 1e-16)

        c0.wait()
        carry = lax.fori_loop(0, GCH, make_edge(stage, b0, 0), carry)
        c1.wait()
        carry = lax.fori_loop(0, GCH, make_edge(stage2, b0 + GCH, GCH), carry)
        return carry

    carry = lax.fori_loop(0, npad // (2 * GCH), pair, carry)

    dcur = carry[0]
    regs = carry[2:]

    @pl.when(dcur < rlen)
    def _():
        for c in range(8):
            acc[dcur, pl.ds(c * 16, 16)] = regs[c]

    pltpu.sync_copy(acc.at[pl.ds(0, RANGE)], xnew_hbm.at[pl.ds(lo, RANGE)])
    pltpu.sync_copy(degr.at[pl.ds(0, RANGE)], deg_hbm.at[pl.ds(lo, RANGE)])


@jax.jit
def _sc2(csrc, cdl, off, cnt, h, beta, s1, mm):
    f = pl.kernel(
        _sc2_body,
        out_type=[
            jax.ShapeDtypeStruct((NPAD, 128), jnp.float32),
            jax.ShapeDtypeStruct((NPAD,), jnp.float32),
        ],
        mesh=_mesh,
        compiler_params=_sc_params,
        scratch_types=[
            pltpu.VMEM((CAP + 16,), jnp.int32),
            pltpu.VMEM((CAP + 16,), jnp.int32),
            pltpu.VMEM((CAP + 16,), jnp.float32),
            pltpu.VMEM((GCH,), jnp.float32),
            pltpu.VMEM((RANGE + 16,), jnp.float32),
            pltpu.VMEM((RANGE + 16,), jnp.float32),
            pltpu.VMEM((RANGE + 16,), jnp.float32),
            pltpu.VMEM((RANGE + 16,), jnp.float32),
            pltpu.VMEM((16 * LS,), jnp.float32),
            pltpu.VMEM((LS + 16,), jnp.int32),
            pltpu.VMEM((RANGE, 128), jnp.float32),
            pltpu.VMEM((GCH, 128), jnp.float32),
            pltpu.VMEM((GCH, 128), jnp.float32),
            pltpu.VMEM((2 * GCH + 16,), jnp.float32),
            pltpu.VMEM((16,), jnp.int32),
            pltpu.SemaphoreType.DMA,
            pltpu.SemaphoreType.DMA,
        ],
    )
    return f(csrc, cdl, off, cnt, h, beta, s1, mm)


def _sc3_body(csrc_hbm, cdl_hbm, cnt_hbm, g1a_hbm, asum_hbm,
              csrc, cdl, gac, as16, red, cntv, sem):
    wid = lax.axis_index("c") * 16 + lax.axis_index("s")
    lo = wid * RANGE

    pltpu.sync_copy(cnt_hbm.at[wid], cntv)
    npad = cntv[...][0]
    pltpu.sync_copy(csrc_hbm.at[wid], csrc.at[pl.ds(0, CAP)])
    pltpu.sync_copy(cdl_hbm.at[wid], cdl.at[pl.ds(0, CAP)])

    def bchunk(g, _):
        pltpu.async_copy(g1a_hbm.at[csrc.at[pl.ds(g * GCH, GCH)]],
                         gac.at[pl.ds(g * GCH, GCH)], sem).wait()
        return 0

    lax.fori_loop(0, npad // GCH, bchunk, 0)

    @pl.loop(0, 16 * LS, step=16)
    def _(i):
        as16[pl.ds(i, 16)] = jnp.zeros((16,), jnp.float32)

    lanes = lax.iota(jnp.int32, 16) * LS

    def veca(j, _):
        dl = cdl[pl.ds(j * 16, 16)]
        ga = gac[pl.ds(j * 16, 16)]
        plsc.addupdate_scatter(as16, [dl + lanes], ga)
        return 0

    lax.fori_loop(0, npad // 16, veca, 0)

    @pl.loop(0, RANGE + 16, step=16)
    def _(i):
        t = jnp.zeros((16,), jnp.float32)
        for l in range(16):
            t = t + as16[pl.ds(l * LS + i, 16)]
        red[pl.ds(i, 16)] = t

    pltpu.sync_copy(red.at[pl.ds(0, RANGE)], asum_hbm.at[pl.ds(lo, RANGE)])


@jax.jit
def _sc3(csrc, cdl, cnt, g1a):
    f = pl.kernel(
        _sc3_body,
        out_type=jax.ShapeDtypeStruct((NPAD,), jnp.float32),
        mesh=_mesh,
        compiler_params=_sc_params,
        scratch_types=[
            pltpu.VMEM((CAP + 16,), jnp.int32),
            pltpu.VMEM((CAP + 16,), jnp.int32),
            pltpu.VMEM((CAP + 16,), jnp.float32),
            pltpu.VMEM((16 * LS,), jnp.float32),
            pltpu.VMEM((RANGE + 16,), jnp.float32),
            pltpu.VMEM((16,), jnp.int32),
            pltpu.SemaphoreType.DMA,
        ],
    )
    return f(csrc, cdl, cnt, g1a)


def _hb_body(x_ref, w_ref, b_ref, wa2_ref, h_ref, beta_ref, bmax_ref):
    i = pl.program_id(0)
    h = jax.nn.relu(
        lax.dot_general(x_ref[...], w_ref[...], (((1,), (0,)), ((), ())),
                        preferred_element_type=jnp.float32) + b_ref[...])
    h_ref[...] = h
    beta = jnp.sum(h * wa2_ref[...], axis=1)
    beta_ref[...] = beta[:, None]
    bm = jnp.max(beta)
    prev = jnp.where(i == 0, jnp.float32(NEG), bmax_ref[...][0, 0])
    bmax_ref[...] = jnp.reshape(jnp.maximum(prev, bm), (1, 1))


@jax.jit
def _k_h(x, W1, b1, wa2):
    blk = 400
    return pl.pallas_call(
        _hb_body,
        grid=(N // blk,),
        in_specs=[
            pl.BlockSpec((blk, 128), lambda i: (i, 0)),
            pl.BlockSpec((128, 128), lambda i: (0, 0)),
            pl.BlockSpec((1, 128), lambda i: (0, 0)),
            pl.BlockSpec((1, 128), lambda i: (0, 0)),
        ],
        out_specs=[
            pl.BlockSpec((blk, 128), lambda i: (i, 0)),
            pl.BlockSpec((blk, 1), lambda i: (i, 0)),
            pl.BlockSpec((1, 1), lambda i: (0, 0)),
        ],
        out_shape=[
            jax.ShapeDtypeStruct((N, 128), jnp.float32),
            jax.ShapeDtypeStruct((N, 1), jnp.float32),
            jax.ShapeDtypeStruct((1, 1), jnp.float32),
        ],
    )(x, W1, b1[None, :], wa2[None, :])


def _alpha_body(xq_ref, u_ref, c0_ref, bmax_ref, s1_ref, mm_ref):
    s1 = jnp.sum(xq_ref[...] * u_ref[...], axis=1) + c0_ref[0, 0]
    s1_ref[...] = s1[:, None]
    z = s1 + bmax_ref[0, 0]
    mm_ref[...] = jnp.where(z > 0, z, 0.2 * z)[:, None]


@jax.jit
def _k_alpha(xq, u, c0, bmax):
    blk = 512
    return pl.pallas_call(
        _alpha_body,
        grid=(NPAD // blk,),
        in_specs=[
            pl.BlockSpec((blk, 128), lambda i: (i, 0)),
            pl.BlockSpec((1, 128), lambda i: (0, 0)),
            pl.BlockSpec((1, 1), lambda i: (0, 0)),
            pl.BlockSpec((1, 1), lambda i: (0, 0)),
        ],
        out_specs=[
            pl.BlockSpec((blk, 1), lambda i: (i, 0)),
            pl.BlockSpec((blk, 1), lambda i: (i, 0)),
        ],
        out_shape=[
            jax.ShapeDtypeStruct((NPAD, 1), jnp.float32),
            jax.ShapeDtypeStruct((NPAD, 1), jnp.float32),
        ],
    )(xq, u[None, :], c0, bmax)


def _g_body(x_ref, wg_ref, bg_ref, g1a_ref, gb_ref, g3_ref):
    x = x_ref[...]
    g1a_ref[...] = (jnp.sum(x * wg_ref[0:1, :], axis=1) + bg_ref[0, 0])[:, None]
    gb_ref[...] = jnp.sum(x * wg_ref[1:2, :], axis=1)[:, None]
    g3_ref[...] = (jnp.sum(x * wg_ref[2:3, :], axis=1) + bg_ref[0, 1])[:, None]


@jax.jit
def _k_g(xnew, wg3x, bgv):
    blk = 512
    return pl.pallas_call(
        _g_body,
        grid=(NPAD // blk,),
        in_specs=[
            pl.BlockSpec((blk, 128), lambda i: (i, 0)),
            pl.BlockSpec((3, 128), lambda i: (0, 0)),
            pl.BlockSpec((1, 2), lambda i: (0, 0)),
        ],
        out_specs=[
            pl.BlockSpec((blk, 1), lambda i: (i, 0)),
            pl.BlockSpec((blk, 1), lambda i: (i, 0)),
            pl.BlockSpec((blk, 1), lambda i: (i, 0)),
        ],
        out_shape=[
            jax.ShapeDtypeStruct((NPAD, 1), jnp.float32),
            jax.ShapeDtypeStruct((NPAD, 1), jnp.float32),
            jax.ShapeDtypeStruct((NPAD, 1), jnp.float32),
        ],
    )(xnew, wg3x, bgv)


def _topk_body(asum_ref, deg_ref, gb_ref, g3_ref, xnew_ref, w2_ref, b2_ref,
               o_ref):
    fit = jax.nn.sigmoid(asum_ref[...] - deg_ref[...] * gb_ref[...]
                         + g3_ref[...])
    idx = lax.broadcasted_iota(jnp.int32, (NPAD,), 0)
    fit = jnp.where(idx < N, fit, -1.0)
    bits = lax.bitcast_convert_type(fit, jnp.int32)

    def sbit(b, thr):
        cand = thr | (1 << b)
        cnt = jnp.sum(jnp.where(bits >= cand, 1, 0))
        return jnp.where(cnt >= K, cand, thr)

    thr = lax.fori_loop(0, 31, lambda i, t: sbit(30 - i, t), 0)

    c_gt = jnp.sum(jnp.where(bits > thr, 1, 0))
    t = K - c_gt
    tie = bits == thr

    def mbit(b, m):
        cand = m | (1 << b)
        g = jnp.sum(jnp.where(tie & (idx < cand), 1, 0))
        return jnp.where(g <= t, cand, m)

    m = lax.fori_loop(0, 14, lambda i, mm: mbit(13 - i, mm), 0)

    sel = (bits > thr) | (tie & (idx < m))
    w = jnp.where(sel, fit, 0.0)
    s = jnp.sum(xnew_ref[...] * w[:, None], axis=0) * (1.0 / K)
    o_ref[...] = (lax.dot_general(s[None, :], w2_ref[...],
                                  (((1,), (0,)), ((), ())),
                                  preferred_element_type=jnp.float32)
                  + b2_ref[...])


@jax.jit
def _k_topk(asum, deg, gb, g3, xnew, W2, b2):
    return pl.pallas_call(
        _topk_body,
        out_shape=jax.ShapeDtypeStruct((1, 64), jnp.float32),
    )(asum, deg, gb, g3, xnew, W2, b2[None, :])


def kernel(x, edge_index, batch, W1, b1, Wp, bp, Wa, ba, Wg1, bg1, Wg2, Wg3, bg3, W2, b2):
    src, dst = edge_index[0], edge_index[1]
    wa1 = Wa[:128, 0]
    wa2 = Wa[128:, 0]
    u = Wp @ wa1
    c0 = jnp.reshape(jnp.dot(bp, wa1) + ba[0], (1, 1))
    wg3x = jnp.stack([Wg1[:, 0], Wg2[:, 0], Wg3[:, 0]], axis=0)
    bgv = jnp.stack([bg1[0], bg3[0]]).reshape(1, 2)

    h, beta, bmax = _k_h(x, W1, b1, wa2)
    csrc, cdl, off, cnt = _sc0(src, dst)
    xq_pad = _sc1(csrc, off, cnt, h)
    s1, mm = _k_alpha(xq_pad, u, c0, bmax)
    beta_pad = jnp.pad(beta.reshape(-1), (0, NPAD - N))
    xnew, deg = _sc2(csrc, cdl, off, cnt, h, beta_pad, s1.reshape(-1),
                     mm.reshape(-1))
    g1a, gb, g3 = _k_g(xnew, wg3x, bgv)
    asum = _sc3(csrc, cdl, cnt, g1a.reshape(-1))
    out = _k_topk(asum, deg, gb.reshape(-1), g3.reshape(-1), xnew, W2, b2)
    return out


# speedup vs baseline: 1.5236x; 1.1020x over previous
import dataclasses
import functools

import jax
import jax.numpy as jnp
import numpy as np
from jax import lax
from jax.experimental import pallas as pl
from jax.experimental.pallas import tpu as pltpu
from jax.experimental.pallas import tpu_sc as plsc

N = 10000
E = 320000
K = 5000
NT = 32
RANGE = 320
NPAD = NT * RANGE
CAP = 12800
ECH = 8000
GCH = 128
NEG = -3.0e38
LS = 384

_mesh = plsc.VectorSubcoreMesh(core_axis_name="c", subcore_axis_name="s")

_sc_params = pltpu.CompilerParams()
if "needs_layout_passes" in pltpu.CompilerParams.__dataclass_fields__:
    _sc_params = dataclasses.replace(_sc_params, needs_layout_passes=False)


def _popc(m):
    return plsc.all_reduce_population_count(m)[0]


def _sc0_body(src_hbm, dst_hbm, csrc_hbm, cdl_hbm, off_hbm, cnt_hbm,
              ebs, ebd, csrc, cdl, tmps, tmpd, lflag, cnt16, cur16, offv,
              cntv, sem0, sem1):
    wid = lax.axis_index("c") * 16 + lax.axis_index("s")
    lo = wid * RANGE
    rlen = jnp.minimum(RANGE, N - lo)

    @pl.loop(0, RANGE, step=16)
    def _(i):
        lflag[pl.ds(i, 16)] = jnp.zeros((16,), jnp.int32)

    ones = jnp.ones((16,), jnp.int32)

    def chunk(ci, pos):
        cs = pltpu.async_copy(src_hbm.at[pl.ds(ci * ECH, ECH)], ebs, sem0)
        cd = pltpu.async_copy(dst_hbm.at[pl.ds(ci * ECH, ECH)], ebd, sem1)
        cs.wait()
        cd.wait()

        def vec(j, pos):
            d = ebd[pl.ds(j * 16, 16)]
            s = ebs[pl.ds(j * 16, 16)]
            dl = d - lo
            m = (dl >= 0) & (dl < rlen) & (pos < CAP - 16)
            mloop = m & (s == d)
            plsc.store_scatter(lflag, [jnp.where(mloop, dl, 0)], ones,
                               mask=mloop)
            plsc.store_compressed(csrc.at[pl.ds(pos, 16)], s, mask=m)
            plsc.store_compressed(cdl.at[pl.ds(pos, 16)], dl, mask=m)
            return pos + _popc(m)

        return lax.fori_loop(0, ECH // 16, vec, pos)

    pos = lax.fori_loop(0, E // ECH, chunk, 0)

    def app(j, pos):
        dl = lax.iota(jnp.int32, 16) + j * 16
        flg = lflag[pl.ds(j * 16, 16)]
        m = (dl < rlen) & (flg == 0) & (pos < CAP - 16)
        plsc.store_compressed(csrc.at[pl.ds(pos, 16)], dl + lo, mask=m)
        plsc.store_compressed(cdl.at[pl.ds(pos, 16)], dl, mask=m)
        return pos + _popc(m)

    pos = lax.fori_loop(0, RANGE // 16, app, pos)

    npad = (pos + (2 * GCH - 1)) & ~(2 * GCH - 1)

    def fill(j, _):
        csrc[pl.ds(pos + j * 16, 16)] = jnp.zeros((16,), jnp.int32)
        cdl[pl.ds(pos + j * 16, 16)] = jnp.full((16,), RANGE, jnp.int32)
        return 0

    lax.fori_loop(0, (npad - pos + 15) // 16, fill, 0)

    lanes = lax.iota(jnp.int32, 16) * LS

    @pl.loop(0, 16 * LS, step=16)
    def _(i):
        cnt16[pl.ds(i, 16)] = jnp.zeros((16,), jnp.int32)

    def vcount(j, _):
        dl = cdl[pl.ds(j * 16, 16)]
        plsc.addupdate_scatter(cnt16, [dl + lanes], ones)
        return 0

    lax.fori_loop(0, npad // 16, vcount, 0)

    def oblk(i, base):
        tot = jnp.zeros((16,), jnp.int32)
        for l in range(16):
            tot = tot + cnt16[pl.ds(l * LS + i * 16, 16)]
        ps = plsc.cumsum(tot)
        offv[pl.ds(i * 16, 16)] = base + (ps - tot)
        return base + ps[15]

    lax.fori_loop(0, LS // 16, oblk, 0)

    @pl.loop(0, LS, step=16)
    def _(i):
        run = offv[pl.ds(i, 16)]
        for l in range(16):
            cur16[pl.ds(l * LS + i, 16)] = run
            run = run + cnt16[pl.ds(l * LS + i, 16)]

    def vscat(j, _):
        dl = cdl[pl.ds(j * 16, 16)]
        s = csrc[pl.ds(j * 16, 16)]
        posv = plsc.load_gather(cur16, [dl + lanes])
        plsc.store_scatter(tmps, [posv], s)
        plsc.store_scatter(tmpd, [posv], dl)
        plsc.store_scatter(cur16, [dl + lanes], posv + 1)
        return 0

    lax.fori_loop(0, npad // 16, vscat, 0)

    pltpu.sync_copy(tmps.at[pl.ds(0, CAP)], csrc_hbm.at[wid])
    pltpu.sync_copy(tmpd.at[pl.ds(0, CAP)], cdl_hbm.at[wid])
    pltpu.sync_copy(offv, off_hbm.at[wid])
    cntv[...] = jnp.full((16,), 0, jnp.int32) + npad
    pltpu.sync_copy(cntv, cnt_hbm.at[wid])


@jax.jit
def _sc0(src, dst):
    f = pl.kernel(
        _sc0_body,
        out_type=[
            jax.ShapeDtypeStruct((NT, CAP), jnp.int32),
            jax.ShapeDtypeStruct((NT, CAP), jnp.int32),
            jax.ShapeDtypeStruct((NT, LS), jnp.int32),
            jax.ShapeDtypeStruct((NT, 16), jnp.int32),
        ],
        mesh=_mesh,
        compiler_params=_sc_params,
        scratch_types=[
            pltpu.VMEM((ECH,), jnp.int32),
            pltpu.VMEM((ECH,), jnp.int32),
            pltpu.VMEM((CAP + 16,), jnp.int32),
            pltpu.VMEM((CAP + 16,), jnp.int32),
            pltpu.VMEM((CAP + 16,), jnp.int32),
            pltpu.VMEM((CAP + 16,), jnp.int32),
            pltpu.VMEM((RANGE,), jnp.int32),
            pltpu.VMEM((16 * LS,), jnp.int32),
            pltpu.VMEM((16 * LS,), jnp.int32),
            pltpu.VMEM((LS,), jnp.int32),
            pltpu.VMEM((16,), jnp.int32),
            pltpu.SemaphoreType.DMA,
            pltpu.SemaphoreType.DMA,
        ],
    )
    return f(src, dst)


def _sc1_body(csrc_hbm, off_hbm, cnt_hbm, h_hbm, xq_hbm,
              csrc, offv, acc, stage, stage2, cntv, sem, sem2):
    wid = lax.axis_index("c") * 16 + lax.axis_index("s")
    lo = wid * RANGE
    rlen = jnp.minimum(RANGE, N - lo)

    pltpu.sync_copy(cnt_hbm.at[wid], cntv)
    npad = cntv[...][0]
    pltpu.sync_copy(csrc_hbm.at[wid], csrc.at[pl.ds(0, CAP)])
    pltpu.sync_copy(off_hbm.at[wid], offv.at[pl.ds(0, LS)])

    @pl.loop(0, 128, step=16)
    def _(c):
        z = jnp.zeros((16,), jnp.float32)

        def zr(r, _):
            acc[r, pl.ds(c, 16)] = z
            return 0

        lax.fori_loop(rlen, RANGE, zr, 0)

    negs = jnp.full((16,), NEG, jnp.float32)

    def make_edge(stg, base):
        def edge(r, carry):
            dcur, nb, r0, r1, r2, r3, r4, r5, r6, r7 = carry
            e = base + r
            flush = (e == nb) & (dcur < rlen)
            regs = [r0, r1, r2, r3, r4, r5, r6, r7]

            @pl.when(flush)
            def _():
                for c in range(8):
                    acc[dcur, pl.ds(c * 16, 16)] = regs[c]

            dcur = jnp.where(flush, dcur + 1, dcur)
            nb = jnp.where(flush, offv[pl.ds(dcur + 1, 16)][0], nb)
            out = []
            for c in range(8):
                b = stg[r, pl.ds(c * 16, 16)]
                out.append(jnp.maximum(jnp.where(flush, negs, regs[c]), b))
            return (dcur, nb, *out)

        return edge

    nb0 = offv[pl.ds(1, 16)][0]
    carry = (jnp.int32(0), nb0, *([negs] * 8))

    def pair(p, carry):
        b0 = p * 2 * GCH
        c0 = pltpu.async_copy(h_hbm.at[csrc.at[pl.ds(b0, GCH)]], stage, sem)
        c1 = pltpu.async_copy(h_hbm.at[csrc.at[pl.ds(b0 + GCH, GCH)]],
                              stage2, sem2)
        c0.wait()
        carry = lax.fori_loop(0, GCH, make_edge(stage, b0), carry)
        c1.wait()
        carry = lax.fori_loop(0, GCH, make_edge(stage2, b0 + GCH), carry)
        return carry

    carry = lax.fori_loop(0, npad // (2 * GCH), pair, carry)

    dcur = carry[0]
    regs = carry[2:]

    @pl.when(dcur < rlen)
    def _():
        for c in range(8):
            acc[dcur, pl.ds(c * 16, 16)] = regs[c]

    pltpu.sync_copy(acc.at[pl.ds(0, RANGE)], xq_hbm.at[pl.ds(lo, RANGE)])


@jax.jit
def _sc1(csrc, off, cnt, h):
    f = pl.kernel(
        _sc1_body,
        out_type=jax.ShapeDtypeStruct((NPAD, 128), jnp.float32),
        mesh=_mesh,
        compiler_params=_sc_params,
        scratch_types=[
            pltpu.VMEM((CAP + 16,), jnp.int32),
            pltpu.VMEM((LS + 16,), jnp.int32),
            pltpu.VMEM((RANGE, 128), jnp.float32),
            pltpu.VMEM((GCH, 128), jnp.float32),
            pltpu.VMEM((GCH, 128), jnp.float32),
            pltpu.VMEM((16,), jnp.int32),
            pltpu.SemaphoreType.DMA,
            pltpu.SemaphoreType.DMA,
        ],
    )
    return f(csrc, off, cnt, h)


def _sc2_body(csrc_hbm, cdl_hbm, off_hbm, cnt_hbm, h_hbm, beta_hbm, s1_hbm,
              mm_hbm, xnew_hbm, deg_hbm,
              csrc, cdl, wv, btc, s1v, mmv, ssr, degr, ss16, offv, acc,
              stage, stage2, wnc, cntv, sem, sem2):
    wid = lax.axis_index("c") * 16 + lax.axis_index("s")
    lo = wid * RANGE
    rlen = jnp.minimum(RANGE, N - lo)

    pltpu.sync_copy(cnt_hbm.at[wid], cntv)
    npad = cntv[...][0]

    pltpu.sync_copy(csrc_hbm.at[wid], csrc.at[pl.ds(0, CAP)])
    pltpu.sync_copy(cdl_hbm.at[wid], cdl.at[pl.ds(0, CAP)])
    pltpu.sync_copy(off_hbm.at[wid], offv.at[pl.ds(0, LS)])
    pltpu.sync_copy(s1_hbm.at[pl.ds(lo, RANGE)], s1v.at[pl.ds(0, RANGE)])
    pltpu.sync_copy(mm_hbm.at[pl.ds(lo, RANGE)], mmv.at[pl.ds(0, RANGE)])
    s1v[pl.ds(RANGE, 16)] = jnp.zeros((16,), jnp.float32)
    mmv[pl.ds(RANGE, 16)] = jnp.zeros((16,), jnp.float32)

    @pl.loop(0, 16 * LS, step=16)
    def _(i):
        ss16[pl.ds(i, 16)] = jnp.zeros((16,), jnp.float32)

    lanes = lax.iota(jnp.int32, 16) * LS

    def bchunk(g, _):
        pltpu.async_copy(beta_hbm.at[csrc.at[pl.ds(g * GCH, GCH)]],
                         btc, sem).wait()

        @pl.loop(0, GCH, step=16)
        def _(j):
            dl = cdl[pl.ds(g * GCH + j, 16)]
            bt = btc[pl.ds(j, 16)]
            a1 = plsc.load_gather(s1v, [dl])
            mm = plsc.load_gather(mmv, [dl])
            z = a1 + bt
            scr = jnp.where(z > 0, z, 0.2 * z)
            w = jnp.exp(scr - mm)
            wv[pl.ds(g * GCH + j, 16)] = w
            plsc.addupdate_scatter(ss16, [dl + lanes], w)

        return 0

    lax.fori_loop(0, npad // GCH, bchunk, 0)

    @pl.loop(0, RANGE + 16, step=16)
    def _(i):
        t = jnp.zeros((16,), jnp.float32)
        for l in range(16):
            t = t + ss16[pl.ds(l * LS + i, 16)]
        ssr[pl.ds(i, 16)] = t

    @pl.loop(0, RANGE, step=16)
    def _(i):
        d0 = offv[pl.ds(i, 16)]
        d1 = offv[pl.ds(i + 1, 16)]
        degr[pl.ds(i, 16)] = (d1 - d0).astype(jnp.float32)

    @pl.loop(0, 128, step=16)
    def _(c):
        z = jnp.zeros((16,), jnp.float32)

        def zr(r, _):
            acc[r, pl.ds(c, 16)] = z
            return 0

        lax.fori_loop(rlen, RANGE, zr, 0)

    zeros = jnp.zeros((16,), jnp.float32)

    def make_edge(stg, base, wbase):
        def edge(r, carry):
            dcur, nb, r0, r1, r2, r3, r4, r5, r6, r7 = carry
            e = base + r
            flush = (e == nb) & (dcur < rlen)
            regs = [r0, r1, r2, r3, r4, r5, r6, r7]

            @pl.when(flush)
            def _():
                for c in range(8):
                    acc[dcur, pl.ds(c * 16, 16)] = regs[c]

            dcur = jnp.where(flush, dcur + 1, dcur)
            nb = jnp.where(flush, offv[pl.ds(dcur + 1, 16)][0], nb)
            wn = wnc[pl.ds(wbase + r, 16)][0]
            out = []
            for c in range(8):
                b = stg[r, pl.ds(c * 16, 16)]
                out.append(jnp.where(flush, zeros, regs[c]) + wn * b)
            return (dcur, nb, *out)

        return edge

    nb0 = offv[pl.ds(1, 16)][0]
    carry = (jnp.int32(0), nb0, *([zeros] * 8))

    def pair(p, carry):
        b0 = p * 2 * GCH
        c0 = pltpu.async_copy(h_hbm.at[csrc.at[pl.ds(b0, GCH)]], stage, sem)
        c1 = pltpu.async_copy(h_hbm.at[csrc.at[pl.ds(b0 + GCH, GCH)]],
                              stage2, sem2)

        @pl.loop(0, 2 * GCH, step=16)
        def _(j):
            dl = cdl[pl.ds(b0 + j, 16)]
            w = wv[pl.ds(b0 + j, 16)]
            ss = plsc.load_gather(ssr, [dl])
            wnc[pl.ds(j, 16)] = w / (ss + 1e-16)

        c0.wait()
        carry = lax.fori_loop(0, GCH, make_edge(stage, b0, 0), carry)
        c1.wait()
        carry = lax.fori_loop(0, GCH, make_edge(stage2, b0 + GCH, GCH), carry)
        return carry

    carry = lax.fori_loop(0, npad // (2 * GCH), pair, carry)

    dcur = carry[0]
    regs = carry[2:]

    @pl.when(dcur < rlen)
    def _():
        for c in range(8):
            acc[dcur, pl.ds(c * 16, 16)] = regs[c]

    pltpu.sync_copy(acc.at[pl.ds(0, RANGE)], xnew_hbm.at[pl.ds(lo, RANGE)])
    pltpu.sync_copy(degr.at[pl.ds(0, RANGE)], deg_hbm.at[pl.ds(lo, RANGE)])


@jax.jit
def _sc2(csrc, cdl, off, cnt, h, beta, s1, mm):
    f = pl.kernel(
        _sc2_body,
        out_type=[
            jax.ShapeDtypeStruct((NPAD, 128), jnp.float32),
            jax.ShapeDtypeStruct((NPAD,), jnp.float32),
        ],
        mesh=_mesh,
        compiler_params=_sc_params,
        scratch_types=[
            pltpu.VMEM((CAP + 16,), jnp.int32),
            pltpu.VMEM((CAP + 16,), jnp.int32),
            pltpu.VMEM((CAP + 16,), jnp.float32),
            pltpu.VMEM((GCH,), jnp.float32),
            pltpu.VMEM((RANGE + 16,), jnp.float32),
            pltpu.VMEM((RANGE + 16,), jnp.float32),
            pltpu.VMEM((RANGE + 16,), jnp.float32),
            pltpu.VMEM((RANGE + 16,), jnp.float32),
            pltpu.VMEM((16 * LS,), jnp.float32),
            pltpu.VMEM((LS + 16,), jnp.int32),
            pltpu.VMEM((RANGE, 128), jnp.float32),
            pltpu.VMEM((GCH, 128), jnp.float32),
            pltpu.VMEM((GCH, 128), jnp.float32),
            pltpu.VMEM((2 * GCH + 16,), jnp.float32),
            pltpu.VMEM((16,), jnp.int32),
            pltpu.SemaphoreType.DMA,
            pltpu.SemaphoreType.DMA,
        ],
    )
    return f(csrc, cdl, off, cnt, h, beta, s1, mm)


def _sc3_body(csrc_hbm, cdl_hbm, cnt_hbm, g1a_hbm, asum_hbm,
              csrc, cdl, gac, as16, red, cntv, sem):
    wid = lax.axis_index("c") * 16 + lax.axis_index("s")
    lo = wid * RANGE

    pltpu.sync_copy(cnt_hbm.at[wid], cntv)
    npad = cntv[...][0]
    pltpu.sync_copy(csrc_hbm.at[wid], csrc.at[pl.ds(0, CAP)])
    pltpu.sync_copy(cdl_hbm.at[wid], cdl.at[pl.ds(0, CAP)])

    def bchunk(g, _):
        pltpu.async_copy(g1a_hbm.at[csrc.at[pl.ds(g * GCH, GCH)]],
                         gac.at[pl.ds(g * GCH, GCH)], sem).wait()
        return 0

    lax.fori_loop(0, npad // GCH, bchunk, 0)

    @pl.loop(0, 16 * LS, step=16)
    def _(i):
        as16[pl.ds(i, 16)] = jnp.zeros((16,), jnp.float32)

    lanes = lax.iota(jnp.int32, 16) * LS

    def veca(j, _):
        dl = cdl[pl.ds(j * 16, 16)]
        ga = gac[pl.ds(j * 16, 16)]
        plsc.addupdate_scatter(as16, [dl + lanes], ga)
        return 0

    lax.fori_loop(0, npad // 16, veca, 0)

    @pl.loop(0, RANGE + 16, step=16)
    def _(i):
        t = jnp.zeros((16,), jnp.float32)
        for l in range(16):
            t = t + as16[pl.ds(l * LS + i, 16)]
        red[pl.ds(i, 16)] = t

    pltpu.sync_copy(red.at[pl.ds(0, RANGE)], asum_hbm.at[pl.ds(lo, RANGE)])


@jax.jit
def _sc3(csrc, cdl, cnt, g1a):
    f = pl.kernel(
        _sc3_body,
        out_type=jax.ShapeDtypeStruct((NPAD,), jnp.float32),
        mesh=_mesh,
        compiler_params=_sc_params,
        scratch_types=[
            pltpu.VMEM((CAP + 16,), jnp.int32),
            pltpu.VMEM((CAP + 16,), jnp.int32),
            pltpu.VMEM((CAP + 16,), jnp.float32),
            pltpu.VMEM((16 * LS,), jnp.float32),
            pltpu.VMEM((RANGE + 16,), jnp.float32),
            pltpu.VMEM((16,), jnp.int32),
            pltpu.SemaphoreType.DMA,
        ],
    )
    return f(csrc, cdl, cnt, g1a)


def _hb_body(x_ref, w_ref, b_ref, wa2_ref, h_ref, beta_ref, bmax_ref):
    i = pl.program_id(0)
    h = jax.nn.relu(
        lax.dot_general(x_ref[...], w_ref[...], (((1,), (0,)), ((), ())),
                        preferred_element_type=jnp.float32) + b_ref[...])
    h_ref[...] = h
    beta = jnp.sum(h * wa2_ref[...], axis=1)
    beta_ref[...] = beta[:, None]
    bm = jnp.max(beta)
    prev = jnp.where(i == 0, jnp.float32(NEG), bmax_ref[...][0, 0])
    bmax_ref[...] = jnp.reshape(jnp.maximum(prev, bm), (1, 1))


@jax.jit
def _k_h(x, W1, b1, wa2):
    blk = 400
    return pl.pallas_call(
        _hb_body,
        grid=(N // blk,),
        in_specs=[
            pl.BlockSpec((blk, 128), lambda i: (i, 0)),
            pl.BlockSpec((128, 128), lambda i: (0, 0)),
            pl.BlockSpec((1, 128), lambda i: (0, 0)),
            pl.BlockSpec((1, 128), lambda i: (0, 0)),
        ],
        out_specs=[
            pl.BlockSpec((blk, 128), lambda i: (i, 0)),
            pl.BlockSpec((blk, 1), lambda i: (i, 0)),
            pl.BlockSpec((1, 1), lambda i: (0, 0)),
        ],
        out_shape=[
            jax.ShapeDtypeStruct((N, 128), jnp.float32),
            jax.ShapeDtypeStruct((N, 1), jnp.float32),
            jax.ShapeDtypeStruct((1, 1), jnp.float32),
        ],
    )(x, W1, b1[None, :], wa2[None, :])


def _alpha_body(xq_ref, u_ref, c0_ref, bmax_ref, s1_ref, mm_ref):
    s1 = jnp.sum(xq_ref[...] * u_ref[...], axis=1) + c0_ref[0, 0]
    s1_ref[...] = s1[:, None]
    z = s1 + bmax_ref[0, 0]
    mm_ref[...] = jnp.where(z > 0, z, 0.2 * z)[:, None]


@jax.jit
def _k_alpha(xq, u, c0, bmax):
    blk = 512
    return pl.pallas_call(
        _alpha_body,
        grid=(NPAD // blk,),
        in_specs=[
            pl.BlockSpec((blk, 128), lambda i: (i, 0)),
            pl.BlockSpec((1, 128), lambda i: (0, 0)),
            pl.BlockSpec((1, 1), lambda i: (0, 0)),
            pl.BlockSpec((1, 1), lambda i: (0, 0)),
        ],
        out_specs=[
            pl.BlockSpec((blk, 1), lambda i: (i, 0)),
            pl.BlockSpec((blk, 1), lambda i: (i, 0)),
        ],
        out_shape=[
            jax.ShapeDtypeStruct((NPAD, 1), jnp.float32),
            jax.ShapeDtypeStruct((NPAD, 1), jnp.float32),
        ],
    )(xq, u[None, :], c0, bmax)


def _g_body(x_ref, wg_ref, bg_ref, g1a_ref, gb_ref, g3_ref):
    x = x_ref[...]
    g1a_ref[...] = (jnp.sum(x * wg_ref[0:1, :], axis=1) + bg_ref[0, 0])[:, None]
    gb_ref[...] = jnp.sum(x * wg_ref[1:2, :], axis=1)[:, None]
    g3_ref[...] = (jnp.sum(x * wg_ref[2:3, :], axis=1) + bg_ref[0, 1])[:, None]


@jax.jit
def _k_g(xnew, wg3x, bgv):
    blk = 512
    return pl.pallas_call(
        _g_body,
        grid=(NPAD // blk,),
        in_specs=[
            pl.BlockSpec((blk, 128), lambda i: (i, 0)),
            pl.BlockSpec((3, 128), lambda i: (0, 0)),
            pl.BlockSpec((1, 2), lambda i: (0, 0)),
        ],
        out_specs=[
            pl.BlockSpec((blk, 1), lambda i: (i, 0)),
            pl.BlockSpec((blk, 1), lambda i: (i, 0)),
            pl.BlockSpec((blk, 1), lambda i: (i, 0)),
        ],
        out_shape=[
            jax.ShapeDtypeStruct((NPAD, 1), jnp.float32),
            jax.ShapeDtypeStruct((NPAD, 1), jnp.float32),
            jax.ShapeDtypeStruct((NPAD, 1), jnp.float32),
        ],
    )(xnew, wg3x, bgv)


def _topk_body(asum_ref, deg_ref, gb_ref, g3_ref, xnew_ref, w2_ref, b2_ref,
               o_ref):
    fit = jax.nn.sigmoid(asum_ref[...] - deg_ref[...] * gb_ref[...]
                         + g3_ref[...])
    idx = lax.broadcasted_iota(jnp.int32, (NPAD,), 0)
    fit = jnp.where(idx < N, fit, -1.0)
    bits = lax.bitcast_convert_type(fit, jnp.int32)

    def sbit(b, thr):
        cand = thr | (1 << b)
        cnt = jnp.sum(jnp.where(bits >= cand, 1, 0))
        return jnp.where(cnt >= K, cand, thr)

    thr = lax.fori_loop(0, 31, lambda i, t: sbit(30 - i, t), 0)

    c_gt = jnp.sum(jnp.where(bits > thr, 1, 0))
    t = K - c_gt
    tie = bits == thr

    def mbit(b, m):
        cand = m | (1 << b)
        g = jnp.sum(jnp.where(tie & (idx < cand), 1, 0))
        return jnp.where(g <= t, cand, m)

    m = lax.fori_loop(0, 14, lambda i, mm: mbit(13 - i, mm), 0)

    sel = (bits > thr) | (tie & (idx < m))
    w = jnp.where(sel, fit, 0.0)
    s = jnp.sum(xnew_ref[...] * w[:, None], axis=0) * (1.0 / K)
    o_ref[...] = (lax.dot_general(s[None, :], w2_ref[...],
                                  (((1,), (0,)), ((), ())),
                                  preferred_element_type=jnp.float32)
                  + b2_ref[...])


@jax.jit
def _k_topk(asum, deg, gb, g3, xnew, W2, b2):
    return pl.pallas_call(
        _topk_body,
        out_shape=jax.ShapeDtypeStruct((1, 64), jnp.float32),
    )(asum, deg, gb, g3, xnew, W2, b2[None, :])


def kernel(x, edge_index, batch, W1, b1, Wp, bp, Wa, ba, Wg1, bg1, Wg2, Wg3, bg3, W2, b2):
    src, dst = edge_index[0], edge_index[1]
    wa1 = Wa[:128, 0]
    wa2 = Wa[128:, 0]
    u = Wp @ wa1
    c0 = jnp.reshape(jnp.dot(bp, wa1) + ba[0], (1, 1))
    wg3x = jnp.stack([Wg1[:, 0], Wg2[:, 0], Wg3[:, 0]], axis=0)
    bgv = jnp.stack([bg1[0], bg3[0]]).reshape(1, 2)

    h, beta, bmax = _k_h(x, W1, b1, wa2)
    csrc, cdl, off, cnt = _sc0(src, dst)
    xq_pad = _sc1(csrc, off, cnt, h)
    s1, mm = _k_alpha(xq_pad, u, c0, bmax)
    beta_pad = jnp.pad(beta.reshape(-1), (0, NPAD - N))
    xnew, deg = _sc2(csrc, cdl, off, cnt, h, beta_pad, s1.reshape(-1),
                     mm.reshape(-1))
    g1a, gb, g3 = _k_g(xnew, wg3x, bgv)
    asum = _sc3(csrc, cdl, cnt, g1a.reshape(-1))
    out = _k_topk(asum, deg, gb.reshape(-1), g3.reshape(-1), xnew, W2, b2)
    return out
